# Initial kernel scaffold; baseline (speedup 1.0000x reference)
import functools

import jax
import jax.numpy as jnp
from jax import lax
from jax.experimental import pallas as pl
from jax.experimental.pallas import tpu as pltpu
from jax.experimental.pallas import tpu_sc as plsc

N = 3_200_000
M = 50_000

NC = 2
NS = 16
NW = NC * NS
L = 16

AP = N // NW
CH = 2_000
NCHUNK = AP // CH
STEPS = CH // L

MROWS = 3_200
MPAD = MROWS * L
RED_CH = 128
NRED = MROWS // RED_CH

_mesh = plsc.VectorSubcoreMesh(core_axis_name="c", subcore_axis_name="s")


def _worker_id():
    return lax.axis_index("s") * NC + lax.axis_index("c")


def _k1_body(q_hbm, idx_hbm, psum_hbm, pcnt_hbm,
             sum_v, cnt_v, idx_s, q_s, rowidx, ssum, scnt):
    cid = lax.axis_index("c")
    sid = lax.axis_index("s")
    wid = sid * NC + cid
    base = wid * AP

    zeros = jnp.zeros((L,), jnp.float32)
    ones = jnp.ones((L,), jnp.float32)
    lane_iota = lax.iota(jnp.int32, L)

    @pl.loop(0, MROWS)
    def _zero(r):
        sum_v[r, :] = zeros
        cnt_v[r, :] = zeros

    @pl.loop(0, MROWS // L)
    def _fill(r):
        rowidx[r // (RED_CH // L), pl.ds((r % (RED_CH // L)) * L, L)] = (
            r * L + lane_iota)

    @pl.when(sid == 0)
    def _():
        pltpu.sync_copy(sum_v, ssum)
        pltpu.sync_copy(cnt_v, scnt)

    plsc.subcore_barrier()

    @pl.loop(0, NCHUNK)
    def _chunk(g):
        off = base + g * CH
        pltpu.sync_copy(idx_hbm.at[pl.ds(off, CH)], idx_s)
        pltpu.sync_copy(q_hbm.at[pl.ds(off, CH)], q_s)

        @pl.loop(0, STEPS)
        def _step(i):
            iv = idx_s[pl.ds(i * L, L)]
            qv = q_s[pl.ds(i * L, L)]
            row = lax.shift_right_logical(iv, 4)
            lane = jnp.bitwise_and(iv, 15)
            plsc.addupdate_scatter(sum_v, [row, lane], qv)
            plsc.addupdate_scatter(cnt_v, [row, lane], ones)

    for j in range(NRED):
        rows = pl.ds(j * RED_CH, RED_CH)
        pltpu.sync_copy(sum_v.at[rows], ssum.at[rowidx.at[j]], add=True)
        pltpu.sync_copy(cnt_v.at[rows], scnt.at[rowidx.at[j]], add=True)

    plsc.subcore_barrier()

    @pl.when(sid == 0)
    def _():
        out_rows = pl.ds(cid * MROWS, MROWS)
        pltpu.sync_copy(ssum, psum_hbm.at[out_rows])
        pltpu.sync_copy(scnt, pcnt_hbm.at[out_rows])


_k1 = functools.partial(
    pl.kernel,
    out_type=(
        jax.ShapeDtypeStruct((NC * MROWS, L), jnp.float32),
        jax.ShapeDtypeStruct((NC * MROWS, L), jnp.float32),
    ),
    mesh=_mesh,
    scratch_types=[
        pltpu.VMEM((MROWS, L), jnp.float32),
        pltpu.VMEM((MROWS, L), jnp.float32),
        pltpu.VMEM((CH,), jnp.int32),
        pltpu.VMEM((CH,), jnp.float32),
        pltpu.VMEM((NRED, RED_CH), jnp.int32),
        pltpu.VMEM_SHARED((MROWS, L), jnp.float32),
        pltpu.VMEM_SHARED((MROWS, L), jnp.float32),
    ],
)(_k1_body)


R2 = MROWS // NW


def _k2_body(psum_hbm, pcnt_hbm, qpad_hbm, corr_hbm,
             s0, s1, c0, c1, qv, cv):
    wid = _worker_id()
    rows = pl.ds(wid * R2, R2)
    pltpu.sync_copy(psum_hbm.at[pl.ds(wid * R2, R2)], s0)
    pltpu.sync_copy(psum_hbm.at[pl.ds(MROWS + wid * R2, R2)], s1)
    pltpu.sync_copy(pcnt_hbm.at[pl.ds(wid * R2, R2)], c0)
    pltpu.sync_copy(pcnt_hbm.at[pl.ds(MROWS + wid * R2, R2)], c1)
    pltpu.sync_copy(qpad_hbm.at[rows], qv)

    @pl.loop(0, R2)
    def _row(r):
        s = s0[r, :] + s1[r, :]
        c = c0[r, :] + c1[r, :]
        cv[r, :] = (qv[r, :] - s) / c

    pltpu.sync_copy(cv, corr_hbm.at[rows])


_k2 = functools.partial(
    pl.kernel,
    out_type=jax.ShapeDtypeStruct((MROWS, L), jnp.float32),
    mesh=_mesh,
    scratch_types=[pltpu.VMEM((R2, L), jnp.float32) for _ in range(6)],
)(_k2_body)


def _k3_body(q_hbm, idx_hbm, corr_hbm, out_hbm, table, idx_s, q_s, o_s):
    wid = _worker_id()
    base = wid * AP

    pltpu.sync_copy(corr_hbm, table)

    @pl.loop(0, NCHUNK)
    def _chunk(g):
        off = base + g * CH
        pltpu.sync_copy(idx_hbm.at[pl.ds(off, CH)], idx_s)
        pltpu.sync_copy(q_hbm.at[pl.ds(off, CH)], q_s)

        @pl.loop(0, STEPS)
        def _step(i):
            iv = idx_s[pl.ds(i * L, L)]
            qv = q_s[pl.ds(i * L, L)]
            o_s[pl.ds(i * L, L)] = qv + plsc.load_gather(table, [iv])

        pltpu.sync_copy(o_s, out_hbm.at[pl.ds(off, CH)])


_k3 = functools.partial(
    pl.kernel,
    out_type=jax.ShapeDtypeStruct((N,), jnp.float32),
    mesh=_mesh,
    scratch_types=[
        pltpu.VMEM((MPAD,), jnp.float32),
        pltpu.VMEM((CH,), jnp.int32),
        pltpu.VMEM((CH,), jnp.float32),
        pltpu.VMEM((CH,), jnp.float32),
    ],
)(_k3_body)


@jax.jit
def kernel(per_atom_charge, atomic_subsystem_indices, per_molecule_charge):
    idx32 = atomic_subsystem_indices.astype(jnp.int32)
    q = per_atom_charge.astype(jnp.float32)
    qpad = jnp.pad(per_molecule_charge.astype(jnp.float32),
                   (0, MPAD - M)).reshape(MROWS, L)
    psum, pcnt = _k1(q, idx32)
    corr = _k2(psum, pcnt, qpad)
    return _k3(q, idx32, corr.reshape(MPAD))


# speedup vs baseline: 285.5144x; 285.5144x over previous
import functools

import jax
import jax.numpy as jnp
from jax import lax
from jax.experimental import pallas as pl
from jax.experimental.pallas import tpu as pltpu
from jax.experimental.pallas import tpu_sc as plsc

N = 3_200_000
M = 50_000

NC = 2
NS = 16
NW = NC * NS
L = 16

AP = N // NW
CH = 2_000
NCHUNK = AP // CH
STEPS = CH // L

MPAD = 51_200
NR = 4
RCH = MPAD // NR
SLR = RCH // NS

_mesh = plsc.VectorSubcoreMesh(core_axis_name="c", subcore_axis_name="s")
_cparams = pltpu.CompilerParams(needs_layout_passes=False)


def _k1_body(q_hbm, idx_hbm, psum_hbm, pcnt_hbm,
             sum_v, cnt_v, idx_s0, idx_s1, q_s0, q_s1, sems):
    bufs = ((idx_s0, q_s0), (idx_s1, q_s1))
    cid = lax.axis_index("c")
    sid = lax.axis_index("s")
    wid = sid * NC + cid
    base = wid * AP

    zeros = jnp.zeros((L,), jnp.float32)
    iota = lax.iota(jnp.int32, L)
    iota_f = iota.astype(jnp.float32)
    shiftidx = jnp.maximum(iota - 1, 0)
    f15 = jnp.full((L,), L - 1, jnp.int32)
    is0 = iota == 0
    sixteen = jnp.full((L,), float(L), jnp.float32)

    def start_in(g, b):
        off = base + g * CH
        ib, qb = bufs[b]
        pltpu.async_copy(idx_hbm.at[pl.ds(off, CH)], ib, sems.at[b, 0])
        pltpu.async_copy(q_hbm.at[pl.ds(off, CH)], qb, sems.at[b, 1])

    def wait_in(g, b):
        off = base + g * CH
        ib, qb = bufs[b]
        pltpu.make_async_copy(
            idx_hbm.at[pl.ds(off, CH)], ib, sems.at[b, 0]).wait()
        pltpu.make_async_copy(
            q_hbm.at[pl.ds(off, CH)], qb, sems.at[b, 1]).wait()

    def process(g, b, carry):
        ib, qb = bufs[b]
        iv0 = ib[pl.ds(0, L)]
        qv0 = qb[pl.ds(0, L)]
        iv1 = ib[pl.ds(min(1, STEPS - 1) * L, L)]
        qv1 = qb[pl.ds(min(1, STEPS - 1) * L, L)]

        def step(i, c):
            cprev, csum, posf, iv, qv, ivn, qvn = c
            nxt = jnp.minimum(i + 2, STEPS - 1) * L
            ivn2 = ib[pl.ds(nxt, L)]
            qvn2 = qb[pl.ds(nxt, L)]
            sh = jnp.take_along_axis(iv, shiftidx, axis=0)
            pv = jnp.where(is0, cprev, sh)
            m = iv != pv
            lc = plsc.cumsum(qv)
            tot = jnp.take_along_axis(lc, f15, axis=0)
            excl = lc - qv
            e = excl + csum
            plsc.addupdate_scatter(sum_v, [pv], e, mask=m)
            plsc.addupdate_scatter(sum_v, [iv], -e, mask=m)
            plsc.addupdate_scatter(cnt_v, [pv], posf, mask=m)
            plsc.addupdate_scatter(cnt_v, [iv], -posf, mask=m)
            cprev = jnp.take_along_axis(iv, f15, axis=0)
            return cprev, csum + tot, posf + sixteen, ivn, qvn, ivn2, qvn2

        out = pl.loop(0, STEPS, init_carry=carry + (iv0, qv0, iv1, qv1),
                      unroll=8)(step)
        return out[:3]

    start_in(0, 0)

    @pl.loop(0, MPAD // L, unroll=8)
    def _zero(r):
        sum_v[pl.ds(r * L, L)] = zeros
        cnt_v[pl.ds(r * L, L)] = zeros

    init = (jnp.full((L,), MPAD - 1, jnp.int32),
            jnp.zeros((L,), jnp.float32),
            iota_f)

    def chunk_pair(gg, carry):
        g0 = 2 * gg
        start_in(g0 + 1, 1)
        wait_in(g0, 0)
        carry = process(g0, 0, carry)

        @pl.when(g0 + 2 < NCHUNK)
        def _():
            start_in(g0 + 2, 0)

        wait_in(g0 + 1, 1)
        return process(g0 + 1, 1, carry)

    cprev, csum, posf = pl.loop(
        0, NCHUNK // 2, init_carry=init)(chunk_pair)

    mask0 = is0
    plsc.addupdate_scatter(sum_v, [cprev], csum, mask=mask0)
    plsc.addupdate_scatter(cnt_v, [cprev], posf, mask=mask0)

    slot = wid * MPAD
    pltpu.async_copy(sum_v, psum_hbm.at[pl.ds(slot, MPAD)], sems.at[0, 0])
    pltpu.async_copy(cnt_v, pcnt_hbm.at[pl.ds(slot, MPAD)], sems.at[0, 1])
    pltpu.make_async_copy(
        sum_v, psum_hbm.at[pl.ds(slot, MPAD)], sems.at[0, 0]).wait()
    pltpu.make_async_copy(
        cnt_v, pcnt_hbm.at[pl.ds(slot, MPAD)], sems.at[0, 1]).wait()


_k1 = functools.partial(
    pl.kernel,
    out_type=(
        jax.ShapeDtypeStruct((NW * MPAD,), jnp.float32),
        jax.ShapeDtypeStruct((NW * MPAD,), jnp.float32),
    ),
    mesh=_mesh,
    compiler_params=_cparams,
    scratch_types=[
        pltpu.VMEM((MPAD,), jnp.float32),
        pltpu.VMEM((MPAD,), jnp.float32),
        pltpu.VMEM((CH,), jnp.int32),
        pltpu.VMEM((CH,), jnp.int32),
        pltpu.VMEM((CH,), jnp.float32),
        pltpu.VMEM((CH,), jnp.float32),
        pltpu.SemaphoreType.DMA((2, 2)),
    ],
)(_k1_body)


E2 = MPAD // NW


def _k2_body(psum_hbm, pcnt_hbm, qpad_hbm, corr_hbm, rds, rdc, qv, cv, sems):
    wid = lax.axis_index("s") * NC + lax.axis_index("c")
    base = wid * E2

    def rd_desc(k):
        return (
            pltpu.make_async_copy(psum_hbm.at[pl.ds(k * MPAD + base, E2)],
                                  rds.at[pl.ds(k * E2, E2)], sems.at[0]),
            pltpu.make_async_copy(pcnt_hbm.at[pl.ds(k * MPAD + base, E2)],
                                  rdc.at[pl.ds(k * E2, E2)], sems.at[1]),
        )

    for k in range(NW):
        ds_, dc_ = rd_desc(k)
        ds_.start()
        dc_.start()
    pltpu.async_copy(qpad_hbm.at[pl.ds(base, E2)], qv, sems.at[2])
    for k in range(NW):
        ds_, dc_ = rd_desc(k)
        ds_.wait()
        dc_.wait()
    pltpu.make_async_copy(
        qpad_hbm.at[pl.ds(base, E2)], qv, sems.at[2]).wait()

    @pl.loop(0, E2 // L)
    def _row(r):
        s = rds[pl.ds(r * L, L)]
        c = rdc[pl.ds(r * L, L)]
        for k in range(1, NW):
            s = s + rds[pl.ds(k * E2 + r * L, L)]
            c = c + rdc[pl.ds(k * E2 + r * L, L)]
        cv[pl.ds(r * L, L)] = (qv[pl.ds(r * L, L)] - s) / c

    pltpu.sync_copy(cv, corr_hbm.at[pl.ds(base, E2)])


_k2 = functools.partial(
    pl.kernel,
    out_type=jax.ShapeDtypeStruct((MPAD,), jnp.float32),
    mesh=_mesh,
    compiler_params=_cparams,
    scratch_types=[
        pltpu.VMEM((NW * E2,), jnp.float32),
        pltpu.VMEM((NW * E2,), jnp.float32),
        pltpu.VMEM((E2,), jnp.float32),
        pltpu.VMEM((E2,), jnp.float32),
        pltpu.SemaphoreType.DMA((3,)),
    ],
)(_k2_body)


def _k3_body(q_hbm, idx_hbm, corr_hbm, out_hbm, table,
             idx_s0, idx_s1, q_s0, q_s1, o_s0, o_s1, sems, osems):
    bufs = ((idx_s0, q_s0, o_s0), (idx_s1, q_s1, o_s1))
    wid = lax.axis_index("s") * NC + lax.axis_index("c")
    base = wid * AP

    iota = lax.iota(jnp.int32, L)
    shiftidx = jnp.maximum(iota - 1, 0)
    is0 = iota == 0
    zero_i = jnp.zeros((L,), jnp.int32)

    pltpu.sync_copy(corr_hbm, table)

    def start_in(g, b):
        off = base + g * CH
        ib, qb, _ = bufs[b]
        pltpu.async_copy(idx_hbm.at[pl.ds(off, CH)], ib, sems.at[b, 0])
        pltpu.async_copy(q_hbm.at[pl.ds(off, CH)], qb, sems.at[b, 1])

    def wait_in(g, b):
        off = base + g * CH
        ib, qb, _ = bufs[b]
        pltpu.make_async_copy(
            idx_hbm.at[pl.ds(off, CH)], ib, sems.at[b, 0]).wait()
        pltpu.make_async_copy(
            q_hbm.at[pl.ds(off, CH)], qb, sems.at[b, 1]).wait()

    def process(g, b):
        ib, qb, ob = bufs[b]
        iv0 = ib[pl.ds(0, L)]
        qv0 = qb[pl.ds(0, L)]
        iv1 = ib[pl.ds(min(1, STEPS - 1) * L, L)]
        cv0 = plsc.load_gather(table, [iv0])

        def _step(i, c):
            qv, cv, ivn = c
            nxt2 = jnp.minimum(i + 2, STEPS - 1) * L
            ivn2 = ib[pl.ds(nxt2, L)]
            qvn = qb[pl.ds(jnp.minimum(i + 1, STEPS - 1) * L, L)]
            cvn = plsc.load_gather(table, [ivn])
            ob[pl.ds(i * L, L)] = qv + cv
            return qvn, cvn, ivn2

        pl.loop(0, STEPS, init_carry=(qv0, cv0, iv1), unroll=8)(_step)

        pltpu.async_copy(ob, out_hbm.at[pl.ds(base + g * CH, CH)],
                         osems.at[b])

    def wait_out(g, b):
        ob = bufs[b][2]
        pltpu.make_async_copy(
            ob, out_hbm.at[pl.ds(base + g * CH, CH)], osems.at[b]
        ).wait()

    start_in(0, 0)

    @pl.loop(0, NCHUNK // 2)
    def _chunk_pair(gg):
        g0 = 2 * gg
        start_in(g0 + 1, 1)
        wait_in(g0, 0)

        @pl.when(g0 >= 2)
        def _():
            wait_out(g0 - 2, 0)

        process(g0, 0)

        @pl.when(g0 + 2 < NCHUNK)
        def _():
            start_in(g0 + 2, 0)

        wait_in(g0 + 1, 1)

        @pl.when(g0 >= 2)
        def _():
            wait_out(g0 - 1, 1)

        process(g0 + 1, 1)

    wait_out(NCHUNK - 2, 0)
    wait_out(NCHUNK - 1, 1)


_k3 = functools.partial(
    pl.kernel,
    out_type=jax.ShapeDtypeStruct((N,), jnp.float32),
    mesh=_mesh,
    compiler_params=_cparams,
    scratch_types=[
        pltpu.VMEM((MPAD,), jnp.float32),
        pltpu.VMEM((CH,), jnp.int32),
        pltpu.VMEM((CH,), jnp.int32),
        pltpu.VMEM((CH,), jnp.float32),
        pltpu.VMEM((CH,), jnp.float32),
        pltpu.VMEM((CH,), jnp.float32),
        pltpu.VMEM((CH,), jnp.float32),
        pltpu.SemaphoreType.DMA((2, 2)),
        pltpu.SemaphoreType.DMA((2,)),
    ],
)(_k3_body)


@jax.jit
def kernel(per_atom_charge, atomic_subsystem_indices, per_molecule_charge):
    idx32 = atomic_subsystem_indices.astype(jnp.int32)
    q = per_atom_charge.astype(jnp.float32)
    qpad = jnp.pad(per_molecule_charge.astype(jnp.float32), (0, MPAD - M))
    psum, pcnt = _k1(q, idx32)
    corr = _k2(psum, pcnt, qpad)
    return _k3(q, idx32, corr)
